# Initial kernel scaffold; baseline (speedup 1.0000x reference)
import functools

import jax
import jax.numpy as jnp
from jax import lax
from jax.experimental import pallas as pl
from jax.experimental.pallas import tpu as pltpu
from jax.experimental.pallas import tpu_sc as plsc

N = 10000
D = 128
B = 512
HID = 64
C = 2

NC = 2
NS = 16
NW = NC * NS
CHUNK = 320
SUB = 64
NSUB = CHUNK // SUB
R = 528
RPW = R // NS
OVERLAP = NW * CHUNK - N

_mesh = plsc.VectorSubcoreMesh(core_axis_name="c", subcore_axis_name="s")


@functools.partial(
    pl.kernel,
    out_type=[
        jax.ShapeDtypeStruct((NC, R, D), jnp.float32),
        jax.ShapeDtypeStruct((NC, R, 16), jnp.float32),
    ],
    mesh=_mesh,
    scratch_types=[
        pltpu.VMEM((CHUNK, D), jnp.float32),
        pltpu.VMEM((NSUB, SUB), jnp.int32),
        pltpu.VMEM((CHUNK, 16), jnp.float32),
        pltpu.VMEM((RPW, D), jnp.float32),
        pltpu.VMEM((RPW, 16), jnp.float32),
        pltpu.VMEM_SHARED((R, D), jnp.float32),
        pltpu.VMEM_SHARED((R, 16), jnp.float32),
    ],
)
def _sc_pool(x_hbm, batch_hbm, sums_hbm, cnt_hbm,
             rows_v, idx_v, ones_v, zsum_v, zcnt_v, sums_sh, cnt_sh):
    c = lax.axis_index("c")
    s = lax.axis_index("s")
    wid = c * NS + s
    base = jnp.minimum(wid * CHUNK, N - CHUNK)

    def fill_ones(i, carry):
        ones_v[i, :] = jnp.ones((16,), jnp.float32)
        return carry

    lax.fori_loop(0, CHUNK, fill_ones, 0)

    def fill_zeros(i, carry):
        for j in range(D // 16):
            zsum_v[i, pl.ds(j * 16, 16)] = jnp.zeros((16,), jnp.float32)
        zcnt_v[i, :] = jnp.zeros((16,), jnp.float32)
        return carry

    lax.fori_loop(0, RPW, fill_zeros, 0)

    pltpu.sync_copy(zsum_v, sums_sh.at[pl.ds(s * RPW, RPW)])
    pltpu.sync_copy(zcnt_v, cnt_sh.at[pl.ds(s * RPW, RPW)])

    pltpu.sync_copy(x_hbm.at[pl.ds(base, CHUNK)], rows_v)
    for j in range(NSUB):
        pltpu.sync_copy(batch_hbm.at[pl.ds(base + j * SUB, SUB)], idx_v.at[j])

    @pl.when(wid == NW - 1)
    def _():
        for k in range(OVERLAP // 16):
            idx_v[k // (SUB // 16), pl.ds((k % (SUB // 16)) * 16, 16)] = (
                jnp.full((16,), B, jnp.int32))

    plsc.subcore_barrier()

    for j in range(NSUB):
        pltpu.sync_copy(rows_v.at[pl.ds(j * SUB, SUB)],
                        sums_sh.at[idx_v.at[j]], add=True)
        pltpu.sync_copy(ones_v.at[pl.ds(j * SUB, SUB)],
                        cnt_sh.at[idx_v.at[j]], add=True)

    plsc.subcore_barrier()

    pltpu.sync_copy(sums_sh.at[pl.ds(s * RPW, RPW)],
                    sums_hbm.at[c, pl.ds(s * RPW, RPW)])
    pltpu.sync_copy(cnt_sh.at[pl.ds(s * RPW, RPW)],
                    cnt_hbm.at[c, pl.ds(s * RPW, RPW)])


def _mlp_body(sums_ref, cnt_ref, w1_ref, b1_ref, w2_ref, b2_ref, out_ref):
    sums = sums_ref[0] + sums_ref[1]
    cnt = cnt_ref[0, :, 0:1] + cnt_ref[1, :, 0:1]
    g = jnp.where(cnt > 0, sums / jnp.maximum(cnt, 1.0), 0.0)
    h = jnp.maximum(
        jnp.dot(g[:B], w1_ref[...], preferred_element_type=jnp.float32)
        + b1_ref[...], 0.0)
    out_ref[...] = (
        jnp.dot(h, w2_ref[...], preferred_element_type=jnp.float32)
        + b2_ref[...])


def kernel(x, edge_index, batch, W1, b1, W2, b2):
    del edge_index
    sums, cnt = _sc_pool(x, batch)
    return pl.pallas_call(
        _mlp_body,
        out_shape=jax.ShapeDtypeStruct((B, C), jnp.float32),
    )(sums, cnt, W1, b1.reshape(1, HID), W2, b2.reshape(1, C))


# speedup vs baseline: 4.0071x; 4.0071x over previous
import functools

import jax
import jax.numpy as jnp
from jax import lax
from jax.experimental import pallas as pl
from jax.experimental.pallas import tpu as pltpu
from jax.experimental.pallas import tpu_sc as plsc

N = 10000
D = 128
B = 512
HID = 64
C = 2

NC = 2
NS = 16
NW = NC * NS
CHUNK = 320
SUB = 64
NSUB = CHUNK // SUB
R = 640
RPW = R // NS
OVERLAP = NW * CHUNK - N

_mesh = plsc.VectorSubcoreMesh(core_axis_name="c", subcore_axis_name="s")


@functools.partial(
    pl.kernel,
    out_type=[
        jax.ShapeDtypeStruct((NC, R, D), jnp.float32),
        jax.ShapeDtypeStruct((NC, R, D), jnp.float32),
    ],
    mesh=_mesh,
    scratch_types=[
        pltpu.VMEM((CHUNK, D), jnp.float32),
        pltpu.VMEM((NSUB, SUB), jnp.int32),
        pltpu.VMEM((CHUNK, D), jnp.float32),
        pltpu.VMEM((RPW, D), jnp.float32),

        pltpu.VMEM_SHARED((R, D), jnp.float32),
        pltpu.VMEM_SHARED((R, D), jnp.float32),
    ],
)
def _sc_pool(x_hbm, batch_hbm, sums_hbm, cnt_hbm,
             rows_v, idx_v, ones_v, zsum_v, sums_sh, cnt_sh):
    c = lax.axis_index("c")
    s = lax.axis_index("s")
    wid = c * NS + s
    base = jnp.minimum(wid * CHUNK, N - CHUNK)

    def fill_ones(i, carry):
        for j in range(D // 16):
            ones_v[i, pl.ds(j * 16, 16)] = jnp.ones((16,), jnp.float32)
        return carry

    lax.fori_loop(0, CHUNK, fill_ones, 0)

    def fill_zeros(i, carry):
        for j in range(D // 16):
            zsum_v[i, pl.ds(j * 16, 16)] = jnp.zeros((16,), jnp.float32)
        return carry

    lax.fori_loop(0, RPW, fill_zeros, 0)

    pltpu.sync_copy(zsum_v, sums_sh.at[pl.ds(s * RPW, RPW)])
    pltpu.sync_copy(zsum_v, cnt_sh.at[pl.ds(s * RPW, RPW)])

    pltpu.sync_copy(x_hbm.at[pl.ds(base, CHUNK)], rows_v)
    for j in range(NSUB):
        pltpu.sync_copy(batch_hbm.at[pl.ds(base + j * SUB, SUB)], idx_v.at[j])

    @pl.when(wid == NW - 1)
    def _():
        for k in range(OVERLAP // 16):
            idx_v[k // (SUB // 16), pl.ds((k % (SUB // 16)) * 16, 16)] = (
                jnp.full((16,), B, jnp.int32))

    plsc.subcore_barrier()

    for j in range(NSUB):
        pltpu.sync_copy(rows_v.at[pl.ds(j * SUB, SUB)],
                        sums_sh.at[idx_v.at[j]], add=True)
        pltpu.sync_copy(ones_v.at[pl.ds(j * SUB, SUB)],
                        cnt_sh.at[idx_v.at[j]], add=True)

    plsc.subcore_barrier()

    pltpu.sync_copy(sums_sh.at[pl.ds(s * RPW, RPW)],
                    sums_hbm.at[c, pl.ds(s * RPW, RPW)])
    pltpu.sync_copy(cnt_sh.at[pl.ds(s * RPW, RPW)],
                    cnt_hbm.at[c, pl.ds(s * RPW, RPW)])


def _mlp_body(sums_ref, cnt_ref, w1_ref, b1_ref, w2_ref, b2_ref, out_ref):
    sums = sums_ref[0] + sums_ref[1]
    cnt = cnt_ref[0, :, 0:1] + cnt_ref[1, :, 0:1]
    g = jnp.where(cnt > 0, sums / jnp.maximum(cnt, 1.0), 0.0)
    h = jnp.maximum(
        jnp.dot(g[:B], w1_ref[...], preferred_element_type=jnp.float32)
        + b1_ref[...], 0.0)
    out_ref[...] = (
        jnp.dot(h, w2_ref[...], preferred_element_type=jnp.float32)
        + b2_ref[...])


def kernel(x, edge_index, batch, W1, b1, W2, b2):
    del edge_index
    sums, cnt = _sc_pool(x, batch)
    return pl.pallas_call(
        _mlp_body,
        out_shape=jax.ShapeDtypeStruct((B, C), jnp.float32),
    )(sums, cnt, W1, b1.reshape(1, HID), W2, b2.reshape(1, C))


# speedup vs baseline: 4.6055x; 1.1493x over previous
import functools

import jax
import jax.numpy as jnp
from jax import lax
from jax.experimental import pallas as pl
from jax.experimental.pallas import tpu as pltpu
from jax.experimental.pallas import tpu_sc as plsc

N = 10000
D = 128
B = 512
HID = 64
C = 2

NC = 2
NS = 16
NW = NC * NS
CHUNK = 320
SUB = 64
NSUB = CHUNK // SUB
R = 640
RPW = R // NS
OVERLAP = NW * CHUNK - N

_mesh = plsc.VectorSubcoreMesh(core_axis_name="c", subcore_axis_name="s")


@functools.partial(
    pl.kernel,
    out_type=[
        jax.ShapeDtypeStruct((NC, R, D), jnp.float32),
        jax.ShapeDtypeStruct((NC, R, D), jnp.float32),
    ],
    mesh=_mesh,
    scratch_types=[
        pltpu.VMEM((CHUNK, D), jnp.float32),
        pltpu.VMEM((NSUB, SUB), jnp.int32),
        pltpu.VMEM((SUB, D), jnp.float32),
        pltpu.VMEM((RPW, D), jnp.float32),

        pltpu.VMEM_SHARED((R, D), jnp.float32),
        pltpu.VMEM_SHARED((R, D), jnp.float32),
        pltpu.SemaphoreType.DMA,
        pltpu.SemaphoreType.DMA,
    ],
)
def _sc_pool(x_hbm, batch_hbm, sums_hbm, cnt_hbm,
             rows_v, idx_v, ones_v, zsum_v, sums_sh, cnt_sh, xsem, bsem):
    c = lax.axis_index("c")
    s = lax.axis_index("s")
    wid = c * NS + s
    base = jnp.minimum(wid * CHUNK, N - CHUNK)

    x_dma = pltpu.async_copy(x_hbm.at[pl.ds(base, CHUNK)], rows_v, xsem)
    b_dmas = [
        pltpu.async_copy(batch_hbm.at[pl.ds(base + j * SUB, SUB)],
                         idx_v.at[j], bsem)
        for j in range(NSUB)
    ]

    def fill_ones(i, carry):
        for j in range(D // 16):
            ones_v[i, pl.ds(j * 16, 16)] = jnp.ones((16,), jnp.float32)
        return carry

    lax.fori_loop(0, SUB, fill_ones, 0)

    def fill_zeros(i, carry):
        for j in range(D // 16):
            zsum_v[i, pl.ds(j * 16, 16)] = jnp.zeros((16,), jnp.float32)
        return carry

    lax.fori_loop(0, RPW, fill_zeros, 0)

    pltpu.sync_copy(zsum_v, sums_sh.at[pl.ds(s * RPW, RPW)])
    pltpu.sync_copy(zsum_v, cnt_sh.at[pl.ds(s * RPW, RPW)])

    for dma in b_dmas:
        dma.wait()

    @pl.when(wid == NW - 1)
    def _():
        for k in range(OVERLAP // 16):
            idx_v[k // (SUB // 16), pl.ds((k % (SUB // 16)) * 16, 16)] = (
                jnp.full((16,), B, jnp.int32))

    plsc.subcore_barrier()
    x_dma.wait()

    for j in range(NSUB):
        pltpu.sync_copy(rows_v.at[pl.ds(j * SUB, SUB)],
                        sums_sh.at[idx_v.at[j]], add=True)
        pltpu.sync_copy(ones_v, cnt_sh.at[idx_v.at[j]], add=True)

    plsc.subcore_barrier()

    pltpu.sync_copy(sums_sh.at[pl.ds(s * RPW, RPW)],
                    sums_hbm.at[c, pl.ds(s * RPW, RPW)])
    pltpu.sync_copy(cnt_sh.at[pl.ds(s * RPW, RPW)],
                    cnt_hbm.at[c, pl.ds(s * RPW, RPW)])


def _mlp_body(sums_ref, cnt_ref, w1_ref, b1_ref, w2_ref, b2_ref, out_ref):
    sums = sums_ref[0] + sums_ref[1]
    cnt = cnt_ref[0, :, 0:1] + cnt_ref[1, :, 0:1]
    g = jnp.where(cnt > 0, sums / jnp.maximum(cnt, 1.0), 0.0)
    h = jnp.maximum(
        jnp.dot(g[:B], w1_ref[...], preferred_element_type=jnp.float32)
        + b1_ref[...], 0.0)
    out_ref[...] = (
        jnp.dot(h, w2_ref[...], preferred_element_type=jnp.float32)
        + b2_ref[...])


def kernel(x, edge_index, batch, W1, b1, W2, b2):
    del edge_index
    sums, cnt = _sc_pool(x, batch)
    return pl.pallas_call(
        _mlp_body,
        out_shape=jax.ShapeDtypeStruct((B, C), jnp.float32),
    )(sums, cnt, W1, b1.reshape(1, HID), W2, b2.reshape(1, C))
